# baseline (speedup 1.0000x reference)
import jax
import jax.numpy as jnp
from jax import lax
from jax.experimental import pallas as pl
from jax.experimental.pallas import tpu as pltpu, tpu_sc as plsc

B, C, H, W = 4, 3, 512, 512
HW = H * W
NC, NS = 2, 16
BPC = B // NC
ROWS_PER_TEC = H // NS
CROWS = 2
CPIX = CROWS * W
NCHUNK = ROWS_PER_TEC // CROWS
NSEG = 4 * CPIX // 128
VREGS = CPIX // 16
ZB = 2048
SLC = C * HW // NS


def _splat_chunk(ci, s, flb, chb, idxst, v0, v1, v2, cst):
    lanef, evi, odi, lo8 = cst

    def vreg_body(v, _):
        off = v * 16
        rowin = v >> 5
        gxb = (v & 31) * 16
        fa = flb[pl.ds(2 * off, 16)]
        fb = flb[pl.ds(2 * off + 16, 16)]
        fxv = jnp.where(lo8, fa[evi], fb[evi])
        fyv = jnp.where(lo8, fa[odi], fb[odi])
        xs = lanef + (gxb + 512).astype(jnp.float32) + fxv
        row = s * ROWS_PER_TEC + ci * CROWS + rowin
        ys = (row + 512).astype(jnp.float32) + fyv
        xt = xs.astype(jnp.int32)
        yt = ys.astype(jnp.int32)
        ax = xs - xt.astype(jnp.float32)
        ay = ys - yt.astype(jnp.float32)
        bx = 1.0 - ax
        by = 1.0 - ay
        x0 = xt - 512
        y0 = yt - 512
        x1 = x0 + 1
        y1 = y0 + 1
        wx0 = jnp.where((x0 >= 0) & (x0 < W), bx, 0.0)
        wx1 = jnp.where((x1 >= 0) & (x1 < W), ax, 0.0)
        wy0 = jnp.where((y0 >= 0) & (y0 < H), by, 0.0)
        wy1 = jnp.where((y1 >= 0) & (y1 < H), ay, 0.0)
        x0c = jnp.clip(x0, 0, W - 1)
        x1c = jnp.clip(x1, 0, W - 1)
        r0 = jnp.clip(y0, 0, H - 1) << 9
        r1 = jnp.clip(y1, 0, H - 1) << 9
        ch0 = chb[pl.ds(off, 16)]
        ch1 = chb[pl.ds(CPIX + off, 16)]
        ch2 = chb[pl.ds(2 * CPIX + off, 16)]
        corners = (
            (r0 + x0c, wx0 * wy0),
            (r0 + x1c, wx1 * wy0),
            (r1 + x0c, wx0 * wy1),
            (r1 + x1c, wx1 * wy1),
        )
        seg = v >> 3
        col = (v & 7) * 16
        for k, (idxk, wk) in enumerate(corners):
            idxst[k][pl.ds(off, 16)] = idxk
            v0[pl.ds(k * CPIX + off, 16)] = ch0 * wk
            v1[pl.ds(k * CPIX + off, 16)] = ch1 * wk
            v2[pl.ds(k * CPIX + off, 16)] = ch2 * wk
        return 0

    lax.fori_loop(0, VREGS, vreg_body, 0)


def _tec_body(im0_hbm, fl_hbm, z_hbm, out_hbm,
              flb, chb, idxst, v0, v1, v2, drb, shared,
              sin, ssc):
    c = lax.axis_index("c")
    s = lax.axis_index("s")

    lane = lax.iota(jnp.int32, 16)
    lanef = lane.astype(jnp.float32)
    evi = (lane & 7) * 2
    odi = evi + 1
    lo8 = lane < 8
    cst = (lanef, evi, odi, lo8)

    def fire_inputs(b, ci, p):
        pix0 = (s * ROWS_PER_TEC + ci * CROWS) * W
        pltpu.async_copy(fl_hbm.at[pl.ds((b * HW + pix0) * 2, 2 * CPIX)],
                         flb[p], sin[p])
        for cc in range(C):
            pltpu.async_copy(
                im0_hbm.at[pl.ds((b * C + cc) * HW + pix0, CPIX)],
                chb[p].at[pl.ds(cc * CPIX, CPIX)], sin[p])

    def drain_inputs(p):
        pltpu.make_async_copy(fl_hbm.at[pl.ds(0, 2 * CPIX)], flb[p],
                              sin[p]).wait()
        pltpu.make_async_copy(im0_hbm.at[pl.ds(0, C * CPIX)], chb[p],
                              sin[p]).wait()

    def fire_scatters(p):
        for cc, vst in enumerate((v0[p], v1[p], v2[p])):
            img = cc * HW
            for k in range(4):
                pltpu.async_copy(
                    vst.at[pl.ds(k * CPIX, CPIX)],
                    shared.at[pl.ds(img, HW)].at[idxst[p][k]],
                    ssc[p], add=True)

    def drain_scatters(p):
        for _ in range(C):
            pltpu.make_async_copy(fl_hbm.at[pl.ds(0, 4 * CPIX)], drb,
                                  ssc[p]).wait()

    def zero_slice():
        pltpu.sync_copy(z_hbm.at[pl.ds(s * SLC, SLC)],
                        shared.at[pl.ds(s * SLC, SLC)])

    for b_loc in range(BPC):
        b = c * BPC + b_loc
        fire_inputs(b, jnp.int32(0), 0)
        if b_loc == 0:
            zero_slice()
        plsc.subcore_barrier()

        def pair_body(pair, _):
            for p in range(2):
                ci = pair * 2 + p
                drain_inputs(p)

                @pl.when(ci < NCHUNK - 1)
                def _():
                    fire_inputs(b, ci + 1, 1 - p)

                @pl.when(pair >= 1)
                def _():
                    drain_scatters(p)

                _splat_chunk(ci, s, flb[p], chb[p],
                             idxst[p], v0[p], v1[p], v2[p], cst)
                fire_scatters(p)
            return 0

        lax.fori_loop(0, NCHUNK // 2, pair_body, 0)
        for p in range(2):
            drain_scatters(p)

        plsc.subcore_barrier()
        pltpu.sync_copy(
            shared.at[pl.ds(s * SLC, SLC)],
            out_hbm.at[pl.ds(b * C * HW + s * SLC, SLC)])
        if b_loc < BPC - 1:
            zero_slice()


def kernel(im0, flow):
    im0r = im0.reshape(B * C * HW)
    fl = flow.reshape(B * HW * 2)
    zs = jnp.zeros((C * HW,), jnp.float32)

    mesh = plsc.VectorSubcoreMesh(core_axis_name="c", subcore_axis_name="s",
                                  num_cores=NC, num_subcores=NS)
    dbl = lambda t: [t, t]
    out = pl.kernel(
        _tec_body,
        out_type=jax.ShapeDtypeStruct((B * C * HW,), jnp.float32),
        mesh=mesh,
        scratch_types=[
            dbl(pltpu.VMEM((2 * CPIX,), jnp.float32)),
            dbl(pltpu.VMEM((C * CPIX,), jnp.float32)),
            dbl([pltpu.VMEM((CPIX,), jnp.int32)] * 4),
            dbl(pltpu.VMEM((4 * CPIX,), jnp.float32)),
            dbl(pltpu.VMEM((4 * CPIX,), jnp.float32)),
            dbl(pltpu.VMEM((4 * CPIX,), jnp.float32)),
            pltpu.VMEM((4 * CPIX,), jnp.float32),
            pltpu.VMEM_SHARED((C * HW,), jnp.float32),
            dbl(pltpu.SemaphoreType.DMA),
            dbl(pltpu.SemaphoreType.DMA),
        ],
    )(im0r, fl, zs)
    return out.reshape(B, C, H, W)


# speedup vs baseline: 8.2700x; 8.2700x over previous
import jax
import jax.numpy as jnp
from jax import lax
from jax.experimental import pallas as pl
from jax.experimental.pallas import tpu as pltpu, tpu_sc as plsc

B, C, H, W = 4, 3, 512, 512
HW = H * W
NC, NS = 2, 16
BPC = B // NC
ROWS_PER_TEC = H // NS
CROWS = 2
CPIX = CROWS * W
NCHUNK = ROWS_PER_TEC // CROWS
NSEG = 4 * CPIX // 128
VREGS = CPIX // 16
ZB = 2048
SLC = C * HW // NS


def _splat_chunk(ci, s, flb, chb, idxst, v0, v1, v2, lanef, lane2):
    def vreg_body(v, _):
        off = v * 16
        rowin = v >> 5
        gxb = (v & 31) * 16
        fxv = flb[pl.ds(off, 16)]
        fyv = flb[pl.ds(CPIX + off, 16)]
        xs = lanef + (gxb + 512).astype(jnp.float32) + fxv
        row = s * ROWS_PER_TEC + ci * CROWS + rowin
        ys = (row + 512).astype(jnp.float32) + fyv
        xt = xs.astype(jnp.int32)
        yt = ys.astype(jnp.int32)
        ax = xs - xt.astype(jnp.float32)
        ay = ys - yt.astype(jnp.float32)
        bx = 1.0 - ax
        by = 1.0 - ay
        x0 = xt - 512
        y0 = yt - 512
        x1 = x0 + 1
        y1 = y0 + 1
        wx0 = jnp.where((x0 >= 0) & (x0 < W), bx, 0.0)
        wx1 = jnp.where((x1 >= 0) & (x1 < W), ax, 0.0)
        wy0 = jnp.where((y0 >= 0) & (y0 < H), by, 0.0)
        wy1 = jnp.where((y1 >= 0) & (y1 < H), ay, 0.0)
        x0c = jnp.clip(x0, 0, W - 1)
        x1c = jnp.clip(x1, 0, W - 1)
        r0 = jnp.clip(y0, 0, H - 1) << 9
        r1 = jnp.clip(y1, 0, H - 1) << 9
        ch0 = chb[pl.ds(off, 16)]
        ch1 = chb[pl.ds(CPIX + off, 16)]
        ch2 = chb[pl.ds(2 * CPIX + off, 16)]
        corners = (
            (r0 + x0c, wx0 * wy0),
            (r0 + x1c, wx1 * wy0),
            (r1 + x0c, wx0 * wy1),
            (r1 + x1c, wx1 * wy1),
        )
        seg = v >> 3
        col = (v & 7) * 16
        for k, (idxk, wk) in enumerate(corners):
            idxst[k][pl.ds(off, 16)] = idxk
            v0[pl.ds(k * CPIX + off, 16)] = ch0 * wk
            v1[pl.ds(k * CPIX + off, 16)] = ch1 * wk
            v2[pl.ds(k * CPIX + off, 16)] = ch2 * wk
        return 0

    lax.fori_loop(0, VREGS, vreg_body, 0)


def _tec_body(im0_hbm, fl_hbm, z_hbm, out_hbm,
              flb, chb, idxst, v0, v1, v2, drb, shared,
              sin, ssc):
    c = lax.axis_index("c")
    s = lax.axis_index("s")

    lane = lax.iota(jnp.int32, 16)
    lanef = lane.astype(jnp.float32)
    lane2 = lane * 2

    def fire_inputs(b, ci, p):
        pix0 = (s * ROWS_PER_TEC + ci * CROWS) * W
        pltpu.async_copy(fl_hbm.at[pl.ds(b * HW + pix0, CPIX)],
                         flb[p].at[pl.ds(0, CPIX)], sin[p])
        pltpu.async_copy(fl_hbm.at[pl.ds(B * HW + b * HW + pix0, CPIX)],
                         flb[p].at[pl.ds(CPIX, CPIX)], sin[p])
        for cc in range(C):
            pltpu.async_copy(
                im0_hbm.at[pl.ds((b * C + cc) * HW + pix0, CPIX)],
                chb[p].at[pl.ds(cc * CPIX, CPIX)], sin[p])

    def drain_inputs(p):
        pltpu.make_async_copy(fl_hbm.at[pl.ds(0, 2 * CPIX)], flb[p],
                              sin[p]).wait()
        pltpu.make_async_copy(im0_hbm.at[pl.ds(0, C * CPIX)], chb[p],
                              sin[p]).wait()

    def fire_scatters(p):
        for cc, vst in enumerate((v0[p], v1[p], v2[p])):
            img = cc * HW
            for k in range(4):
                pltpu.async_copy(
                    vst.at[pl.ds(k * CPIX, CPIX)],
                    shared.at[pl.ds(img, HW)].at[idxst[p][k]],
                    ssc[p], add=True)

    def drain_scatters(p):
        for _ in range(C):
            pltpu.make_async_copy(fl_hbm.at[pl.ds(0, 4 * CPIX)], drb,
                                  ssc[p]).wait()

    def zero_slice():
        pltpu.sync_copy(z_hbm.at[pl.ds(s * SLC, SLC)],
                        shared.at[pl.ds(s * SLC, SLC)])

    for b_loc in range(BPC):
        b = c * BPC + b_loc
        fire_inputs(b, jnp.int32(0), 0)
        if b_loc == 0:
            zero_slice()
        plsc.subcore_barrier()

        def pair_body(pair, _):
            for p in range(2):
                ci = pair * 2 + p
                drain_inputs(p)

                @pl.when(ci < NCHUNK - 1)
                def _():
                    fire_inputs(b, ci + 1, 1 - p)

                @pl.when(pair >= 1)
                def _():
                    drain_scatters(p)

                _splat_chunk(ci, s, flb[p], chb[p],
                             idxst[p], v0[p], v1[p], v2[p], lanef, lane2)
                fire_scatters(p)
            return 0

        lax.fori_loop(0, NCHUNK // 2, pair_body, 0)
        for p in range(2):
            drain_scatters(p)

        plsc.subcore_barrier()
        pltpu.sync_copy(
            shared.at[pl.ds(s * SLC, SLC)],
            out_hbm.at[pl.ds(b * C * HW + s * SLC, SLC)])
        if b_loc < BPC - 1:
            zero_slice()


def kernel(im0, flow):
    im0r = im0.reshape(B * C * HW)
    fl = jnp.concatenate(
        [flow[..., 0].reshape(B * HW), flow[..., 1].reshape(B * HW)]) + 0.0
    zs = jnp.zeros((C * HW,), jnp.float32)

    mesh = plsc.VectorSubcoreMesh(core_axis_name="c", subcore_axis_name="s",
                                  num_cores=NC, num_subcores=NS)
    dbl = lambda t: [t, t]
    out = pl.kernel(
        _tec_body,
        out_type=jax.ShapeDtypeStruct((B * C * HW,), jnp.float32),
        mesh=mesh,
        scratch_types=[
            dbl(pltpu.VMEM((2 * CPIX,), jnp.float32)),
            dbl(pltpu.VMEM((C * CPIX,), jnp.float32)),
            dbl([pltpu.VMEM((CPIX,), jnp.int32)] * 4),
            dbl(pltpu.VMEM((4 * CPIX,), jnp.float32)),
            dbl(pltpu.VMEM((4 * CPIX,), jnp.float32)),
            dbl(pltpu.VMEM((4 * CPIX,), jnp.float32)),
            pltpu.VMEM((4 * CPIX,), jnp.float32),
            pltpu.VMEM_SHARED((C * HW,), jnp.float32),
            dbl(pltpu.SemaphoreType.DMA),
            dbl(pltpu.SemaphoreType.DMA),
        ],
    )(im0r, fl, zs)
    return out.reshape(B, C, H, W)


# speedup vs baseline: 8.2759x; 1.0007x over previous
import jax
import jax.numpy as jnp
from jax import lax
from jax.experimental import pallas as pl
from jax.experimental.pallas import tpu as pltpu, tpu_sc as plsc

B, C, H, W = 4, 3, 512, 512
HW = H * W
NC, NS = 2, 16
BPC = B // NC
ROWS_PER_TEC = H // NS
CROWS = 2
CPIX = CROWS * W
NCHUNK = ROWS_PER_TEC // CROWS
NSEG = 4 * CPIX // 128
VREGS = CPIX // 16
ZB = 2048
SLC = C * HW // NS


def _splat_chunk(ci, s, flb, chb, idxst, v0, v1, v2, lanef, lane2):
    def vreg_body(v, _):
        off = v * 16
        rowin = v >> 5
        gxb = (v & 31) * 16
        xs = flb[pl.ds(off, 16)]
        ys = flb[pl.ds(CPIX + off, 16)]
        xt = xs.astype(jnp.int32)
        yt = ys.astype(jnp.int32)
        ax = xs - xt.astype(jnp.float32)
        ay = ys - yt.astype(jnp.float32)
        bx = 1.0 - ax
        by = 1.0 - ay
        x0 = xt - 512
        y0 = yt - 512
        x1 = x0 + 1
        y1 = y0 + 1
        wx0 = jnp.where((x0 >= 0) & (x0 < W), bx, 0.0)
        wx1 = jnp.where((x1 >= 0) & (x1 < W), ax, 0.0)
        wy0 = jnp.where((y0 >= 0) & (y0 < H), by, 0.0)
        wy1 = jnp.where((y1 >= 0) & (y1 < H), ay, 0.0)
        x0c = jnp.clip(x0, 0, W - 1)
        x1c = jnp.clip(x1, 0, W - 1)
        r0 = jnp.clip(y0, 0, H - 1) << 9
        r1 = jnp.clip(y1, 0, H - 1) << 9
        ch0 = chb[pl.ds(off, 16)]
        ch1 = chb[pl.ds(CPIX + off, 16)]
        ch2 = chb[pl.ds(2 * CPIX + off, 16)]
        corners = (
            (r0 + x0c, wx0 * wy0),
            (r0 + x1c, wx1 * wy0),
            (r1 + x0c, wx0 * wy1),
            (r1 + x1c, wx1 * wy1),
        )
        seg = v >> 3
        col = (v & 7) * 16
        for k, (idxk, wk) in enumerate(corners):
            idxst[k][pl.ds(off, 16)] = idxk
            v0[pl.ds(k * CPIX + off, 16)] = ch0 * wk
            v1[pl.ds(k * CPIX + off, 16)] = ch1 * wk
            v2[pl.ds(k * CPIX + off, 16)] = ch2 * wk
        return 0

    lax.fori_loop(0, VREGS, vreg_body, 0)


def _tec_body(im0_hbm, fl_hbm, z_hbm, out_hbm,
              flb, chb, idxst, v0, v1, v2, drb, shared,
              sin, ssc):
    c = lax.axis_index("c")
    s = lax.axis_index("s")

    lane = lax.iota(jnp.int32, 16)
    lanef = lane.astype(jnp.float32)
    lane2 = lane * 2

    def fire_inputs(b, ci, p):
        pix0 = (s * ROWS_PER_TEC + ci * CROWS) * W
        pltpu.async_copy(fl_hbm.at[pl.ds(b * HW + pix0, CPIX)],
                         flb[p].at[pl.ds(0, CPIX)], sin[p])
        pltpu.async_copy(fl_hbm.at[pl.ds(B * HW + b * HW + pix0, CPIX)],
                         flb[p].at[pl.ds(CPIX, CPIX)], sin[p])
        for cc in range(C):
            pltpu.async_copy(
                im0_hbm.at[pl.ds((b * C + cc) * HW + pix0, CPIX)],
                chb[p].at[pl.ds(cc * CPIX, CPIX)], sin[p])

    def drain_inputs(p):
        pltpu.make_async_copy(fl_hbm.at[pl.ds(0, 2 * CPIX)], flb[p],
                              sin[p]).wait()
        pltpu.make_async_copy(im0_hbm.at[pl.ds(0, C * CPIX)], chb[p],
                              sin[p]).wait()

    def fire_scatters(p):
        for cc, vst in enumerate((v0[p], v1[p], v2[p])):
            img = cc * HW
            for k in range(4):
                pltpu.async_copy(
                    vst.at[pl.ds(k * CPIX, CPIX)],
                    shared.at[pl.ds(img, HW)].at[idxst[p][k]],
                    ssc[p], add=True)

    def drain_scatters(p):
        for _ in range(C):
            pltpu.make_async_copy(fl_hbm.at[pl.ds(0, 4 * CPIX)], drb,
                                  ssc[p]).wait()

    def zero_slice():
        pltpu.sync_copy(z_hbm.at[pl.ds(s * SLC, SLC)],
                        shared.at[pl.ds(s * SLC, SLC)])

    for b_loc in range(BPC):
        b = c * BPC + b_loc
        fire_inputs(b, jnp.int32(0), 0)
        if b_loc == 0:
            zero_slice()
        plsc.subcore_barrier()

        def pair_body(pair, _):
            for p in range(2):
                ci = pair * 2 + p
                drain_inputs(p)

                @pl.when(ci < NCHUNK - 1)
                def _():
                    fire_inputs(b, ci + 1, 1 - p)

                @pl.when(pair >= 1)
                def _():
                    drain_scatters(p)

                _splat_chunk(ci, s, flb[p], chb[p],
                             idxst[p], v0[p], v1[p], v2[p], lanef, lane2)
                fire_scatters(p)
            return 0

        lax.fori_loop(0, NCHUNK // 2, pair_body, 0)
        for p in range(2):
            drain_scatters(p)

        plsc.subcore_barrier()
        pltpu.sync_copy(
            shared.at[pl.ds(s * SLC, SLC)],
            out_hbm.at[pl.ds(b * C * HW + s * SLC, SLC)])
        if b_loc < BPC - 1:
            zero_slice()


def kernel(im0, flow):
    im0r = im0.reshape(B * C * HW)
    gx = (jnp.arange(W, dtype=jnp.float32) + 512.0)[None, None, :]
    gy = (jnp.arange(H, dtype=jnp.float32) + 512.0)[None, :, None]
    fl = jnp.concatenate(
        [(flow[..., 0] + gx).reshape(B * HW),
         (flow[..., 1] + gy).reshape(B * HW)])
    zs = jnp.zeros((C * HW,), jnp.float32)

    mesh = plsc.VectorSubcoreMesh(core_axis_name="c", subcore_axis_name="s",
                                  num_cores=NC, num_subcores=NS)
    dbl = lambda t: [t, t]
    out = pl.kernel(
        _tec_body,
        out_type=jax.ShapeDtypeStruct((B * C * HW,), jnp.float32),
        mesh=mesh,
        scratch_types=[
            dbl(pltpu.VMEM((2 * CPIX,), jnp.float32)),
            dbl(pltpu.VMEM((C * CPIX,), jnp.float32)),
            dbl([pltpu.VMEM((CPIX,), jnp.int32)] * 4),
            dbl(pltpu.VMEM((4 * CPIX,), jnp.float32)),
            dbl(pltpu.VMEM((4 * CPIX,), jnp.float32)),
            dbl(pltpu.VMEM((4 * CPIX,), jnp.float32)),
            pltpu.VMEM((4 * CPIX,), jnp.float32),
            pltpu.VMEM_SHARED((C * HW,), jnp.float32),
            dbl(pltpu.SemaphoreType.DMA),
            dbl(pltpu.SemaphoreType.DMA),
        ],
    )(im0r, fl, zs)
    return out.reshape(B, C, H, W)


# speedup vs baseline: 8.2765x; 1.0001x over previous
import jax
import jax.numpy as jnp
from jax import lax
from jax.experimental import pallas as pl
from jax.experimental.pallas import tpu as pltpu, tpu_sc as plsc

B, C, H, W = 4, 3, 512, 512
HW = H * W
NC, NS = 2, 16
BPC = B // NC
ROWS_PER_TEC = H // NS
CROWS = 2
CPIX = CROWS * W
NCHUNK = ROWS_PER_TEC // CROWS
VREGS = CPIX // 16
SLC = C * HW // NS


def _splat_chunk(flb, chb, idxst, v0, v1, v2):
    def vreg_body(v, _):
        off = v * 16
        xs = flb[pl.ds(off, 16)]
        ys = flb[pl.ds(CPIX + off, 16)]
        xt = xs.astype(jnp.int32)
        yt = ys.astype(jnp.int32)
        ax = xs - xt.astype(jnp.float32)
        ay = ys - yt.astype(jnp.float32)
        bx = 1.0 - ax
        by = 1.0 - ay
        x0 = xt - 512
        y0 = yt - 512
        x1 = x0 + 1
        y1 = y0 + 1
        wx0 = jnp.where((x0 >= 0) & (x0 < W), bx, 0.0)
        wx1 = jnp.where((x1 >= 0) & (x1 < W), ax, 0.0)
        wy0 = jnp.where((y0 >= 0) & (y0 < H), by, 0.0)
        wy1 = jnp.where((y1 >= 0) & (y1 < H), ay, 0.0)
        x0c = jnp.clip(x0, 0, W - 1)
        x1c = jnp.clip(x1, 0, W - 1)
        r0 = jnp.clip(y0, 0, H - 1) << 9
        r1 = jnp.clip(y1, 0, H - 1) << 9
        ch0 = chb[pl.ds(off, 16)]
        ch1 = chb[pl.ds(CPIX + off, 16)]
        ch2 = chb[pl.ds(2 * CPIX + off, 16)]
        corners = (
            (r0 + x0c, wx0 * wy0),
            (r0 + x1c, wx1 * wy0),
            (r1 + x0c, wx0 * wy1),
            (r1 + x1c, wx1 * wy1),
        )
        for k, (idxk, wk) in enumerate(corners):
            idxst[k][pl.ds(off, 16)] = idxk
            v0[pl.ds(k * CPIX + off, 16)] = ch0 * wk
            v1[pl.ds(k * CPIX + off, 16)] = ch1 * wk
            v2[pl.ds(k * CPIX + off, 16)] = ch2 * wk
        return 0

    lax.fori_loop(0, VREGS, vreg_body, 0)


def _tec_body(im0_hbm, fl_hbm, z_hbm, out_hbm,
              flb, chb, idxst, v0, v1, v2, drb, shared,
              sin, ssc):
    c = lax.axis_index("c")
    s = lax.axis_index("s")

    def fire_inputs(b, ci, p):
        pix0 = (s * ROWS_PER_TEC + ci * CROWS) * W
        pltpu.async_copy(fl_hbm.at[pl.ds(b * HW + pix0, CPIX)],
                         flb[p].at[pl.ds(0, CPIX)], sin[p])
        pltpu.async_copy(fl_hbm.at[pl.ds(B * HW + b * HW + pix0, CPIX)],
                         flb[p].at[pl.ds(CPIX, CPIX)], sin[p])
        for cc in range(C):
            pltpu.async_copy(
                im0_hbm.at[pl.ds((b * C + cc) * HW + pix0, CPIX)],
                chb[p].at[pl.ds(cc * CPIX, CPIX)], sin[p])

    def drain_inputs(p):
        pltpu.make_async_copy(fl_hbm.at[pl.ds(0, 2 * CPIX)], flb[p],
                              sin[p]).wait()
        pltpu.make_async_copy(im0_hbm.at[pl.ds(0, C * CPIX)], chb[p],
                              sin[p]).wait()

    def fire_scatters(p):
        for cc, vst in enumerate((v0[p], v1[p], v2[p])):
            img = cc * HW
            for k in range(4):
                pltpu.async_copy(
                    vst.at[pl.ds(k * CPIX, CPIX)],
                    shared.at[pl.ds(img, HW)].at[idxst[p][k]],
                    ssc[p], add=True)

    def drain_scatters(p):
        for _ in range(C):
            pltpu.make_async_copy(fl_hbm.at[pl.ds(0, 4 * CPIX)], drb,
                                  ssc[p]).wait()

    def zero_slice():
        pltpu.sync_copy(z_hbm.at[pl.ds(s * SLC, SLC)],
                        shared.at[pl.ds(s * SLC, SLC)])

    for b_loc in range(BPC):
        b = c * BPC + b_loc
        fire_inputs(b, jnp.int32(0), 0)
        if b_loc == 0:
            zero_slice()
        plsc.subcore_barrier()

        def pair_body(pair, _):
            for p in range(2):
                ci = pair * 2 + p
                drain_inputs(p)

                @pl.when(ci < NCHUNK - 1)
                def _():
                    fire_inputs(b, ci + 1, 1 - p)

                @pl.when(pair >= 1)
                def _():
                    drain_scatters(p)

                _splat_chunk(flb[p], chb[p],
                             idxst[p], v0[p], v1[p], v2[p])
                fire_scatters(p)
            return 0

        lax.fori_loop(0, NCHUNK // 2, pair_body, 0)
        for p in range(2):
            drain_scatters(p)

        plsc.subcore_barrier()
        pltpu.sync_copy(
            shared.at[pl.ds(s * SLC, SLC)],
            out_hbm.at[pl.ds(b * C * HW + s * SLC, SLC)])
        if b_loc < BPC - 1:
            zero_slice()


def kernel(im0, flow):
    im0r = im0.reshape(B * C * HW)
    gx = (jnp.arange(W, dtype=jnp.float32) + 512.0)[None, None, :]
    gy = (jnp.arange(H, dtype=jnp.float32) + 512.0)[None, :, None]
    fl = jnp.concatenate(
        [(flow[..., 0] + gx).reshape(B * HW),
         (flow[..., 1] + gy).reshape(B * HW)])
    zs = jnp.zeros((C * HW,), jnp.float32)

    mesh = plsc.VectorSubcoreMesh(core_axis_name="c", subcore_axis_name="s",
                                  num_cores=NC, num_subcores=NS)
    dbl = lambda t: [t, t]
    out = pl.kernel(
        _tec_body,
        out_type=jax.ShapeDtypeStruct((B * C * HW,), jnp.float32),
        mesh=mesh,
        scratch_types=[
            dbl(pltpu.VMEM((2 * CPIX,), jnp.float32)),
            dbl(pltpu.VMEM((C * CPIX,), jnp.float32)),
            dbl([pltpu.VMEM((CPIX,), jnp.int32)] * 4),
            dbl(pltpu.VMEM((4 * CPIX,), jnp.float32)),
            dbl(pltpu.VMEM((4 * CPIX,), jnp.float32)),
            dbl(pltpu.VMEM((4 * CPIX,), jnp.float32)),
            pltpu.VMEM((4 * CPIX,), jnp.float32),
            pltpu.VMEM_SHARED((C * HW,), jnp.float32),
            dbl(pltpu.SemaphoreType.DMA),
            dbl(pltpu.SemaphoreType.DMA),
        ],
    )(im0r, fl, zs)
    return out.reshape(B, C, H, W)


# speedup vs baseline: 8.2939x; 1.0021x over previous
import jax
import jax.numpy as jnp
from jax import lax
from jax.experimental import pallas as pl
from jax.experimental.pallas import tpu as pltpu, tpu_sc as plsc

B, C, H, W = 4, 3, 512, 512
HW = H * W
NC, NS = 2, 16
BPC = B // NC
ROWS_PER_TEC = H // NS
CROWS = 2
CPIX = CROWS * W
NCHUNK = ROWS_PER_TEC // CROWS
VREGS = CPIX // 16
SLC = C * HW // NS


def _splat_chunk(flb, chb, idxst, v0, v1, v2):
    def vreg_body(v, _):
        off = v * 16
        xs = flb[pl.ds(off, 16)]
        ys = flb[pl.ds(CPIX + off, 16)]
        xt = xs.astype(jnp.int32)
        yt = ys.astype(jnp.int32)
        ax = xs - xt.astype(jnp.float32)
        ay = ys - yt.astype(jnp.float32)
        bx = 1.0 - ax
        by = 1.0 - ay
        x0 = xt - 512
        y0 = yt - 512
        x1 = x0 + 1
        y1 = y0 + 1
        wx0 = jnp.where((x0 >= 0) & (x0 < W), bx, 0.0)
        wx1 = jnp.where((x1 >= 0) & (x1 < W), ax, 0.0)
        wy0 = jnp.where((y0 >= 0) & (y0 < H), by, 0.0)
        wy1 = jnp.where((y1 >= 0) & (y1 < H), ay, 0.0)
        x0c = jnp.clip(x0, 0, W - 1)
        x1c = jnp.clip(x1, 0, W - 1)
        r0 = jnp.clip(y0, 0, H - 1) << 9
        r1 = jnp.clip(y1, 0, H - 1) << 9
        ch0 = chb[pl.ds(off, 16)]
        ch1 = chb[pl.ds(CPIX + off, 16)]
        ch2 = chb[pl.ds(2 * CPIX + off, 16)]
        corners = (
            (r0 + x0c, wx0 * wy0),
            (r0 + x1c, wx1 * wy0),
            (r1 + x0c, wx0 * wy1),
            (r1 + x1c, wx1 * wy1),
        )
        for k, (idxk, wk) in enumerate(corners):
            idxst[pl.ds(k * CPIX + off, 16)] = idxk
            v0[pl.ds(k * CPIX + off, 16)] = ch0 * wk
            v1[pl.ds(k * CPIX + off, 16)] = ch1 * wk
            v2[pl.ds(k * CPIX + off, 16)] = ch2 * wk
        return 0

    lax.fori_loop(0, VREGS, vreg_body, 0)


def _tec_body(im0_hbm, fl_hbm, z_hbm, out_hbm,
              flb, chb, idxst, v0, v1, v2, drb, shared,
              sin, ssc):
    c = lax.axis_index("c")
    s = lax.axis_index("s")

    def fire_inputs(b, ci, p):
        pix0 = (s * ROWS_PER_TEC + ci * CROWS) * W
        pltpu.async_copy(fl_hbm.at[pl.ds(b * HW + pix0, CPIX)],
                         flb[p].at[pl.ds(0, CPIX)], sin[p])
        pltpu.async_copy(fl_hbm.at[pl.ds(B * HW + b * HW + pix0, CPIX)],
                         flb[p].at[pl.ds(CPIX, CPIX)], sin[p])
        for cc in range(C):
            pltpu.async_copy(
                im0_hbm.at[pl.ds((b * C + cc) * HW + pix0, CPIX)],
                chb[p].at[pl.ds(cc * CPIX, CPIX)], sin[p])

    def drain_inputs(p):
        pltpu.make_async_copy(fl_hbm.at[pl.ds(0, 2 * CPIX)], flb[p],
                              sin[p]).wait()
        pltpu.make_async_copy(im0_hbm.at[pl.ds(0, C * CPIX)], chb[p],
                              sin[p]).wait()

    def fire_scatters(p):
        for cc, vst in enumerate((v0[p], v1[p], v2[p])):
            img = cc * HW
            pltpu.async_copy(
                vst,
                shared.at[pl.ds(img, HW)].at[idxst[p]],
                ssc[p], add=True)

    def drain_scatters(p):
        for _ in range(C):
            pltpu.make_async_copy(fl_hbm.at[pl.ds(0, 4 * CPIX)], drb,
                                  ssc[p]).wait()

    def zero_slice():
        pltpu.sync_copy(z_hbm.at[pl.ds(s * SLC, SLC)],
                        shared.at[pl.ds(s * SLC, SLC)])

    for b_loc in range(BPC):
        b = c * BPC + b_loc
        fire_inputs(b, jnp.int32(0), 0)
        if b_loc == 0:
            zero_slice()
        plsc.subcore_barrier()

        def pair_body(pair, _):
            for p in range(2):
                ci = pair * 2 + p
                drain_inputs(p)

                @pl.when(ci < NCHUNK - 1)
                def _():
                    fire_inputs(b, ci + 1, 1 - p)

                @pl.when(pair >= 1)
                def _():
                    drain_scatters(p)

                _splat_chunk(flb[p], chb[p],
                             idxst[p], v0[p], v1[p], v2[p])
                fire_scatters(p)
            return 0

        lax.fori_loop(0, NCHUNK // 2, pair_body, 0)
        for p in range(2):
            drain_scatters(p)

        plsc.subcore_barrier()
        pltpu.sync_copy(
            shared.at[pl.ds(s * SLC, SLC)],
            out_hbm.at[pl.ds(b * C * HW + s * SLC, SLC)])
        if b_loc < BPC - 1:
            zero_slice()


def kernel(im0, flow):
    im0r = im0.reshape(B * C * HW)
    gx = (jnp.arange(W, dtype=jnp.float32) + 512.0)[None, None, :]
    gy = (jnp.arange(H, dtype=jnp.float32) + 512.0)[None, :, None]
    fl = jnp.concatenate(
        [(flow[..., 0] + gx).reshape(B * HW),
         (flow[..., 1] + gy).reshape(B * HW)])
    zs = jnp.zeros((C * HW,), jnp.float32)

    mesh = plsc.VectorSubcoreMesh(core_axis_name="c", subcore_axis_name="s",
                                  num_cores=NC, num_subcores=NS)
    dbl = lambda t: [t, t]
    out = pl.kernel(
        _tec_body,
        out_type=jax.ShapeDtypeStruct((B * C * HW,), jnp.float32),
        mesh=mesh,
        scratch_types=[
            dbl(pltpu.VMEM((2 * CPIX,), jnp.float32)),
            dbl(pltpu.VMEM((C * CPIX,), jnp.float32)),
            dbl(pltpu.VMEM((4 * CPIX,), jnp.int32)),
            dbl(pltpu.VMEM((4 * CPIX,), jnp.float32)),
            dbl(pltpu.VMEM((4 * CPIX,), jnp.float32)),
            dbl(pltpu.VMEM((4 * CPIX,), jnp.float32)),
            pltpu.VMEM((4 * CPIX,), jnp.float32),
            pltpu.VMEM_SHARED((C * HW,), jnp.float32),
            dbl(pltpu.SemaphoreType.DMA),
            dbl(pltpu.SemaphoreType.DMA),
        ],
    )(im0r, fl, zs)
    return out.reshape(B, C, H, W)
